# baseline (speedup 1.0000x reference)
import functools

import jax
import jax.numpy as jnp
from jax import lax
from jax.experimental import pallas as pl
from jax.experimental.pallas import tpu as pltpu
from jax.experimental.pallas import tpu_sc as plsc

_INFO = plsc.get_sparse_core_info()
_NC, _NS = _INFO.num_cores, _INFO.num_subcores
_NW = _NC * _NS
_L = 16

_CHUNK = 512


@functools.lru_cache(maxsize=None)
def _build(B, V, D):
    b_per_w = B // _NW
    n_chunks = b_per_w // _CHUNK
    assert B % _NW == 0 and b_per_w % _CHUNK == 0 and n_chunks % 2 == 0
    H = D // 2
    mesh = plsc.VectorSubcoreMesh(core_axis_name="c", subcore_axis_name="s")

    @functools.partial(
        pl.kernel,
        mesh=mesh,
        out_type=jax.ShapeDtypeStruct((B, 2, H), jnp.float32),
        scratch_types=[
            pltpu.VMEM((b_per_w,), jnp.int32),
            [pltpu.VMEM((_CHUNK,), jnp.int32) for _ in range(4)],
            [pltpu.VMEM((_CHUNK, H), jnp.float32) for _ in range(4)],
            [pltpu.SemaphoreType.DMA for _ in range(2)],
            [pltpu.SemaphoreType.DMA for _ in range(2)],
        ],
        compiler_params=pltpu.CompilerParams(use_tc_tiling_on_sc=False),
    )
    def gather_kernel(tab_hbm, idx_hbm, out_hbm, raw_v, eo_idx, eo_buf, gsem, osem):
        wid = lax.axis_index("s") * _NC + lax.axis_index("c")
        base = wid * b_per_w

        pltpu.sync_copy(idx_hbm.at[pl.ds(base, b_per_w)], raw_v)

        def prep_idx(c, q):
            def blk(k, carry):
                v = raw_v[pl.ds(c * _CHUNK + k * _L, _L)]
                e = v << 1
                eo_idx[2 * q][pl.ds(k * _L, _L)] = e
                eo_idx[2 * q + 1][pl.ds(k * _L, _L)] = e + 1
                return carry

            lax.fori_loop(0, _CHUNK // _L, blk, 0, unroll=4)

        def gather_start(q):
            pltpu.async_copy(tab_hbm.at[eo_idx[2 * q]], eo_buf[2 * q], gsem[q])
            pltpu.async_copy(tab_hbm.at[eo_idx[2 * q + 1]], eo_buf[2 * q + 1], gsem[q])

        def gather_wait(q):
            pltpu.make_async_copy(tab_hbm.at[eo_idx[2 * q]], eo_buf[2 * q], gsem[q]).wait()
            pltpu.make_async_copy(tab_hbm.at[eo_idx[2 * q + 1]], eo_buf[2 * q + 1], gsem[q]).wait()

        def wb_copies(c, q):
            return (
                pltpu.make_async_copy(
                    eo_buf[2 * q],
                    out_hbm.at[pl.ds(base + c * _CHUNK, _CHUNK), 0],
                    osem[q],
                ),
                pltpu.make_async_copy(
                    eo_buf[2 * q + 1],
                    out_hbm.at[pl.ds(base + c * _CHUNK, _CHUNK), 1],
                    osem[q],
                ),
            )

        def wb_start(c, q):
            for cp in wb_copies(c, q):
                cp.start()

        def wb_wait(c, q):
            for cp in wb_copies(c, q):
                cp.wait()

        prep_idx(0, 0)
        gather_start(0)
        prep_idx(1, 1)

        def step_one(c, q, j):
            gather_wait(q)

            @pl.when(j > 0)
            def _():
                wb_wait(c - 2, q)

            wb_start(c, q)

            @pl.when(c + 2 < n_chunks)
            def _():
                prep_idx(c + 2, q)

            @pl.when(c + 1 < n_chunks)
            def _():
                gather_start(1 - q)

        def step(j, carry):
            step_one(2 * j, 0, j)
            step_one(2 * j + 1, 1, j)
            return carry

        lax.fori_loop(0, n_chunks // 2, step, 0, unroll=False)
        wb_wait(n_chunks - 2, 0)
        wb_wait(n_chunks - 1, 1)

    return gather_kernel


def kernel(x, peso):
    B0, S = x.shape
    V, D = peso.shape
    tab = peso.reshape(2 * V, D // 2)
    flat_idx = x.reshape(B0 * S)
    out = _build(B0 * S, V, D)(tab, flat_idx)
    return out.reshape(B0, S, D)
